# Initial kernel scaffold; baseline (speedup 1.0000x reference)
import jax
import jax.numpy as jnp
from jax.experimental import pallas as pl


def kernel(feat_A, feat_B, edge_AB, edge_BA, Wproj_A, Wproj_B, lin1_W, lin1_b, lin2_W, lin2_b, Wcoef):
    raise NotImplementedError("write your pallas kernel here")


# speedup vs baseline: 5.9871x; 5.9871x over previous
import functools

import jax
import jax.numpy as jnp
from jax import lax
from jax.experimental import pallas as pl
from jax.experimental.pallas import tpu as pltpu
from jax.experimental.pallas import tpu_sc as plsc

N = 10000
D = 128
E = 256000
NCORES = 2
NSUB = 16
ROWS_PER_TILE = 640
ROWS_LAST = N - (NSUB - 1) * ROWS_PER_TILE
EDGES_PER_TILE = E // NSUB
CHUNK = 128
NCHUNK = EDGES_PER_TILE // CHUNK

@functools.lru_cache(maxsize=1)
def _get_spmm_pair():
    mesh = plsc.VectorSubcoreMesh(
        core_axis_name="c", subcore_axis_name="s",
        num_cores=NCORES, num_subcores=NSUB)

    @functools.partial(
        pl.kernel,
        out_type=jax.ShapeDtypeStruct((NCORES, N, D), jnp.float32),
        mesh=mesh,
        scratch_types=[
            pltpu.VMEM_SHARED((N, D), jnp.float32),
            pltpu.VMEM((CHUNK,), jnp.int32),
            pltpu.VMEM((CHUNK,), jnp.int32),
            pltpu.VMEM((CHUNK, D), jnp.float32),
            pltpu.SemaphoreType.DMA,
        ],
    )
    def _spmm_pair(x_hbm, edges_hbm, zeros_hbm, out_hbm, acc, si, di, rows, sem):
        cid = lax.axis_index("c")
        sid = lax.axis_index("s")
        rbase = sid * ROWS_PER_TILE

        @pl.when(sid < NSUB - 1)
        def _():
            pltpu.sync_copy(zeros_hbm, acc.at[pl.ds(rbase, ROWS_PER_TILE)])

        @pl.when(sid == NSUB - 1)
        def _():
            pltpu.sync_copy(zeros_hbm.at[pl.ds(0, ROWS_LAST)],
                            acc.at[pl.ds(rbase, ROWS_LAST)])

        plsc.subcore_barrier()

        def chunk_body(i, carry):
            ebase = sid * EDGES_PER_TILE + i * CHUNK
            pltpu.sync_copy(edges_hbm.at[2 * cid + 1, 0, pl.ds(ebase, CHUNK)], si)
            pltpu.sync_copy(edges_hbm.at[2 * cid, 0, pl.ds(ebase, CHUNK)], di)
            pltpu.async_copy(x_hbm.at[si], rows, sem).wait()
            pltpu.sync_copy(rows, acc.at[di], add=True)
            return carry

        lax.fori_loop(0, NCHUNK, chunk_body, 0)
        plsc.subcore_barrier()

        @pl.when(sid < NSUB - 1)
        def _():
            pltpu.sync_copy(acc.at[pl.ds(rbase, ROWS_PER_TILE)],
                            out_hbm.at[cid, pl.ds(rbase, ROWS_PER_TILE)])

        @pl.when(sid == NSUB - 1)
        def _():
            pltpu.sync_copy(acc.at[pl.ds(rbase, ROWS_LAST)],
                            out_hbm.at[cid, pl.ds(rbase, ROWS_LAST)])

    return _spmm_pair


BLK = 1000
NBLK = N // BLK


def _pre_body(feats_ref, wa_ref, wb_ref, w1_ref, b1_ref, o_ref):
    pid = pl.program_id(0)
    w = jnp.where(pid * BLK < N // 2, wa_ref[...], wb_ref[...])
    h = jnp.dot(feats_ref[...], w, preferred_element_type=jnp.float32)
    z = jnp.maximum(
        jnp.dot(h, w1_ref[...], preferred_element_type=jnp.float32)
        + b1_ref[...], 0.0)
    mean = jnp.mean(z, axis=1, keepdims=True)
    zc = z - mean
    std = jnp.sqrt(jnp.sum(zc * zc, axis=1, keepdims=True) / (D - 1))
    o_ref[...] = jnp.where(std > 0.0, zc / std, 0.0)


def _pre(feats, wa, wb, w1, b1):
    return pl.pallas_call(
        _pre_body,
        grid=(NBLK,),
        in_specs=[
            pl.BlockSpec((BLK, D), lambda i: (i, 0)),
            pl.BlockSpec((D, D), lambda i: (0, 0)),
            pl.BlockSpec((D, D), lambda i: (0, 0)),
            pl.BlockSpec((D, D), lambda i: (0, 0)),
            pl.BlockSpec((1, D), lambda i: (0, 0)),
        ],
        out_specs=pl.BlockSpec((BLK, D), lambda i: (i, 0)),
        out_shape=jax.ShapeDtypeStruct((N, D), jnp.float32),
    )(feats, wa, wb, w1, b1)


def _comb_body(wc_ref, x_ref, p1_ref, p2_ref, o_ref):
    o_ref[...] = (wc_ref[0] * x_ref[...]
                  + wc_ref[1] * p1_ref[0, 0] + wc_ref[2] * p1_ref[0, 1]
                  + wc_ref[3] * p2_ref[0, 0] + wc_ref[4] * p2_ref[0, 1])


def _combine(wcoef, x, pair1, pair2):
    stacked1 = pair1.reshape(1, 2, N, D)
    stacked2 = pair2.reshape(1, 2, N, D)
    return pl.pallas_call(
        _comb_body,
        grid=(NBLK,),
        in_specs=[
            pl.BlockSpec(memory_space=pltpu.SMEM),
            pl.BlockSpec((BLK, D), lambda i: (i, 0)),
            pl.BlockSpec((1, 2, BLK, D), lambda i: (0, 0, i, 0)),
            pl.BlockSpec((1, 2, BLK, D), lambda i: (0, 0, i, 0)),
        ],
        out_specs=pl.BlockSpec((BLK, D), lambda i: (i, 0)),
        out_shape=jax.ShapeDtypeStruct((N, D), jnp.float32),
    )(wcoef, x, stacked1, stacked2)


def _fin_body(wc_ref, y_ref, p1_ref, p2_ref, w2_ref, b2_ref, o_ref):
    z = (wc_ref[0] * y_ref[...]
         + wc_ref[1] * p1_ref[0, 1] + wc_ref[2] * p1_ref[0, 0]
         + wc_ref[3] * p2_ref[0, 1] + wc_ref[4] * p2_ref[0, 0])
    o_ref[...] = (jnp.dot(z, w2_ref[...], preferred_element_type=jnp.float32)
                  + b2_ref[...])


def _final(wcoef, y, pair1, pair2, w2, b2):
    nc = w2.shape[1]
    return pl.pallas_call(
        _fin_body,
        grid=(NBLK,),
        in_specs=[
            pl.BlockSpec(memory_space=pltpu.SMEM),
            pl.BlockSpec((BLK, D), lambda i: (i, 0)),
            pl.BlockSpec((1, 2, BLK, D), lambda i: (0, 0, i, 0)),
            pl.BlockSpec((1, 2, BLK, D), lambda i: (0, 0, i, 0)),
            pl.BlockSpec((D, nc), lambda i: (0, 0)),
            pl.BlockSpec((1, nc), lambda i: (0, 0)),
        ],
        out_specs=pl.BlockSpec((BLK, nc), lambda i: (i, 0)),
        out_shape=jax.ShapeDtypeStruct((N, nc), jnp.float32),
    )(wcoef, y, pair1.reshape(1, 2, N, D), pair2.reshape(1, 2, N, D),
      w2, b2)


def kernel(feat_A, feat_B, edge_AB, edge_BA, Wproj_A, Wproj_B,
           lin1_W, lin1_b, lin2_W, lin2_b, Wcoef):
    feats = jnp.concatenate([feat_A, feat_B], axis=0)
    edges = jnp.stack([edge_AB, edge_BA]).astype(jnp.int32).reshape(4, 1, E)
    zeros = jnp.zeros((ROWS_PER_TILE, D), jnp.float32)

    x = _pre(feats, Wproj_A, Wproj_B, lin1_W, lin1_b.reshape(1, D))

    spmm_pair = _get_spmm_pair()
    pa = spmm_pair(x, edges, zeros)
    pb = spmm_pair(pa[1], edges, zeros)
    y = _combine(Wcoef, x, pa, pb)

    pc = spmm_pair(y, edges, zeros)
    pd = spmm_pair(pc[0], edges, zeros)
    return _final(Wcoef, y, pc, pd, lin2_W, lin2_b.reshape(1, -1))


# speedup vs baseline: 34.2422x; 5.7193x over previous
import functools

import jax
import jax.numpy as jnp
from jax import lax
from jax.experimental import pallas as pl
from jax.experimental.pallas import tpu as pltpu
from jax.experimental.pallas import tpu_sc as plsc

N = 10000
D = 128
F = 16
E = 256000
NCORES = 2
NSUB = 16
ROWS_PER_TILE = 640
ROWS_LAST = N - (NSUB - 1) * ROWS_PER_TILE
EDGES_PER_TILE = E // NSUB
CHUNK = 128
NCHUNK = EDGES_PER_TILE // CHUNK
RROW = 6
G = 3
RSRC = 6
RDST = 12
UNROLL = 12


@functools.lru_cache(maxsize=1)
def _get_spmm_pair():
    mesh = plsc.VectorSubcoreMesh(
        core_axis_name="c", subcore_axis_name="s",
        num_cores=NCORES, num_subcores=NSUB)

    @functools.partial(
        pl.kernel,
        out_type=jax.ShapeDtypeStruct((NCORES, N, F), jnp.float32),
        mesh=mesh,
        scratch_types=(
            [pltpu.VMEM_SHARED((N, F), jnp.float32)]
            + [pltpu.VMEM((CHUNK,), jnp.int32) for _ in range(RSRC)]
            + [pltpu.VMEM((CHUNK,), jnp.int32) for _ in range(RDST)]
            + [pltpu.VMEM((CHUNK, F), jnp.float32) for _ in range(RROW)]
            + [pltpu.SemaphoreType.DMA for _ in range(RSRC + RDST + 2 * RROW)]
        ),
        compiler_params=pltpu.CompilerParams(use_tc_tiling_on_sc=False),
    )
    def _spmm_pair(x_hbm, edges_hbm, zeros_hbm, out_hbm, acc, *bufs):
        sidx = bufs[:RSRC]
        didx = bufs[RSRC:RSRC + RDST]
        rows = bufs[RSRC + RDST:RSRC + RDST + RROW]
        sems = bufs[RSRC + RDST + RROW:]
        semsi = sems[:RSRC]
        semdi = sems[RSRC:RSRC + RDST]
        semr = sems[RSRC + RDST:RSRC + RDST + RROW]
        semw = sems[RSRC + RDST + RROW:]
        cid = lax.axis_index("c")
        sid = lax.axis_index("s")
        rbase = sid * ROWS_PER_TILE
        tb = sid * NCHUNK

        def pref_s(c, s):
            pltpu.async_copy(edges_hbm.at[2 * cid + 1, tb + c, 0],
                             sidx[s], semsi[s])

        def pref_d(c, d):
            pltpu.async_copy(edges_hbm.at[2 * cid, tb + c, 0],
                             didx[d], semdi[d])

        def wait_si(s):
            pltpu.make_async_copy(edges_hbm.at[0, 0, 0],
                                  sidx[s], semsi[s]).wait()

        def wait_di(d):
            pltpu.make_async_copy(edges_hbm.at[0, 0, 0],
                                  didx[d], semdi[d]).wait()

        def wait_rows(sem, r):
            pltpu.make_async_copy(x_hbm.at[pl.ds(0, CHUNK)],
                                  rows[r], sem[r]).wait()

        def gather(s, r):
            pltpu.async_copy(x_hbm.at[sidx[s]], rows[r], semr[r])

        def scatter(r, d):
            pltpu.async_copy(rows[r], acc.at[didx[d]], semw[r], add=True)

        for k in range(RSRC):
            pref_s(k, k)
        for k in range(G * 3):
            pref_d(k, k)

        @pl.when(sid < NSUB - 1)
        def _():
            pltpu.sync_copy(zeros_hbm, acc.at[pl.ds(rbase, ROWS_PER_TILE)])

        @pl.when(sid == NSUB - 1)
        def _():
            pltpu.sync_copy(zeros_hbm.at[pl.ds(0, ROWS_LAST)],
                            acc.at[pl.ds(rbase, ROWS_LAST)])

        plsc.subcore_barrier()
        for k in range(G):
            wait_si(k)
            gather(k, k)

        def step(c, u, first=False, pref=(True, True), do_gather=True):
            if not first:
                wait_rows(semw, (u + G) % RROW)
            if do_gather:
                wait_si((u + G) % RSRC)
                gather((u + G) % RSRC, (u + G) % RROW)
            wait_rows(semr, u % RROW)
            wait_di(u % RDST)
            scatter(u % RROW, u % RDST)
            if pref[0]:
                pref_s(c + RSRC, u % RSRC)
            if pref[1]:
                pref_d(c + RDST - G, (u - G) % RDST)

        for c in range(G):
            step(c, c, first=True)

        nsteady = (NCHUNK - 2 * G) // UNROLL

        def chunk_body(g, carry):
            for u0 in range(UNROLL):
                step(UNROLL * g + u0 + G, (u0 + G) % UNROLL)
            return carry

        lax.fori_loop(0, nsteady, chunk_body, 0)

        for c in range(G + UNROLL * nsteady, NCHUNK):
            step(c, c % UNROLL,
                 pref=(c + RSRC < NCHUNK, c + RDST - G < NCHUNK),
                 do_gather=c + G < NCHUNK)
        for c in range(NCHUNK - G, NCHUNK):
            wait_rows(semw, c % RROW)
        plsc.subcore_barrier()

        @pl.when(sid < NSUB - 1)
        def _():
            pltpu.sync_copy(acc.at[pl.ds(rbase, ROWS_PER_TILE)],
                            out_hbm.at[cid, pl.ds(rbase, ROWS_PER_TILE)])

        @pl.when(sid == NSUB - 1)
        def _():
            pltpu.sync_copy(acc.at[pl.ds(rbase, ROWS_LAST)],
                            out_hbm.at[cid, pl.ds(rbase, ROWS_LAST)])

    return _spmm_pair


BLK = 1000
NBLK = N // BLK


def _pre_body(feats_ref, wa_ref, wb_ref, w1_ref, b1_ref, w2_ref, o_ref):
    pid = pl.program_id(0)
    w = jnp.where(pid * BLK < N // 2, wa_ref[...], wb_ref[...])
    h = jnp.dot(feats_ref[...], w, preferred_element_type=jnp.float32)
    z = jnp.maximum(
        jnp.dot(h, w1_ref[...], preferred_element_type=jnp.float32)
        + b1_ref[...], 0.0)
    mean = jnp.mean(z, axis=1, keepdims=True)
    zc = z - mean
    std = jnp.sqrt(jnp.sum(zc * zc, axis=1, keepdims=True) / (D - 1))
    xn = jnp.where(std > 0.0, zc / std, 0.0)
    o_ref[...] = jnp.dot(xn, w2_ref[...], preferred_element_type=jnp.float32)


def _pre(feats, wa, wb, w1, b1, w2):
    return pl.pallas_call(
        _pre_body,
        grid=(NBLK,),
        in_specs=[
            pl.BlockSpec((BLK, D), lambda i: (i, 0)),
            pl.BlockSpec((D, D), lambda i: (0, 0)),
            pl.BlockSpec((D, D), lambda i: (0, 0)),
            pl.BlockSpec((D, D), lambda i: (0, 0)),
            pl.BlockSpec((1, D), lambda i: (0, 0)),
            pl.BlockSpec((D, F), lambda i: (0, 0)),
        ],
        out_specs=pl.BlockSpec((BLK, F), lambda i: (i, 0)),
        out_shape=jax.ShapeDtypeStruct((N, F), jnp.float32),
    )(feats, wa, wb, w1, b1, w2)


def _comb_body(wc_ref, x_ref, p1_ref, p2_ref, o_ref):
    o_ref[...] = (wc_ref[0] * x_ref[...]
                  + wc_ref[1] * p1_ref[0, 0] + wc_ref[2] * p1_ref[0, 1]
                  + wc_ref[3] * p2_ref[0, 0] + wc_ref[4] * p2_ref[0, 1])


def _fin_body(wc_ref, y_ref, p1_ref, p2_ref, b2_ref, o_ref):
    o_ref[...] = (wc_ref[0] * y_ref[...]
                  + wc_ref[1] * p1_ref[0, 1] + wc_ref[2] * p1_ref[0, 0]
                  + wc_ref[3] * p2_ref[0, 1] + wc_ref[4] * p2_ref[0, 0]
                  + b2_ref[...])


def _poly_combine(body, wcoef, x, pair1, pair2, *extra):
    extra_specs = [pl.BlockSpec((1, F), lambda i: (0, 0))] * len(extra)
    return pl.pallas_call(
        body,
        grid=(NBLK,),
        in_specs=[
            pl.BlockSpec(memory_space=pltpu.SMEM),
            pl.BlockSpec((BLK, F), lambda i: (i, 0)),
            pl.BlockSpec((1, 2, BLK, F), lambda i: (0, 0, i, 0)),
            pl.BlockSpec((1, 2, BLK, F), lambda i: (0, 0, i, 0)),
        ] + extra_specs,
        out_specs=pl.BlockSpec((BLK, F), lambda i: (i, 0)),
        out_shape=jax.ShapeDtypeStruct((N, F), jnp.float32),
    )(wcoef, x, pair1.reshape(1, 2, N, F), pair2.reshape(1, 2, N, F), *extra)


def kernel(feat_A, feat_B, edge_AB, edge_BA, Wproj_A, Wproj_B,
           lin1_W, lin1_b, lin2_W, lin2_b, Wcoef):
    feats = jnp.concatenate([feat_A, feat_B], axis=0)
    edges = (jnp.stack([edge_AB, edge_BA]).astype(jnp.int32)
             .reshape(4, NSUB * NCHUNK, 1, CHUNK))
    zeros = jnp.zeros((ROWS_PER_TILE, F), jnp.float32)

    z = _pre(feats, Wproj_A, Wproj_B, lin1_W, lin1_b.reshape(1, D), lin2_W)

    spmm_pair = _get_spmm_pair()
    pa = spmm_pair(z, edges, zeros)
    pb = spmm_pair(pa[1], edges, zeros)
    y = _poly_combine(_comb_body, Wcoef, z, pa, pb)

    pc = spmm_pair(y, edges, zeros)
    pd = spmm_pair(pc[0], edges, zeros)
    return _poly_combine(_fin_body, Wcoef, y, pc, pd, lin2_b.reshape(1, F))


# speedup vs baseline: 38.1570x; 1.1143x over previous
import functools

import jax
import jax.numpy as jnp
from jax import lax
from jax.experimental import pallas as pl
from jax.experimental.pallas import tpu as pltpu
from jax.experimental.pallas import tpu_sc as plsc

N = 10000
D = 128
F = 16
E = 256000
NCORES = 2
NSUB = 16
ROWS_PER_TILE = 640
ROWS_LAST = N - (NSUB - 1) * ROWS_PER_TILE
EDGES_PER_TILE = E // NSUB
CHUNK = 640
NCHUNK = EDGES_PER_TILE // CHUNK
RROW = 6
G = 3
RSRC = 6
RDST = 12
UNROLL = 12


@functools.lru_cache(maxsize=1)
def _get_spmm_pair():
    mesh = plsc.VectorSubcoreMesh(
        core_axis_name="c", subcore_axis_name="s",
        num_cores=NCORES, num_subcores=NSUB)

    @functools.partial(
        pl.kernel,
        out_type=jax.ShapeDtypeStruct((NCORES, N, F), jnp.float32),
        mesh=mesh,
        scratch_types=(
            [pltpu.VMEM_SHARED((N, F), jnp.float32)]
            + [pltpu.VMEM((CHUNK,), jnp.int32) for _ in range(RSRC)]
            + [pltpu.VMEM((CHUNK,), jnp.int32) for _ in range(RDST)]
            + [pltpu.VMEM((CHUNK, F), jnp.float32) for _ in range(RROW)]
            + [pltpu.SemaphoreType.DMA for _ in range(RSRC + RDST + 2 * RROW)]
        ),
        compiler_params=pltpu.CompilerParams(use_tc_tiling_on_sc=False),
    )
    def _spmm_pair(x_hbm, edges_hbm, zeros_hbm, out_hbm, acc, *bufs):
        sidx = bufs[:RSRC]
        didx = bufs[RSRC:RSRC + RDST]
        rows = bufs[RSRC + RDST:RSRC + RDST + RROW]
        sems = bufs[RSRC + RDST + RROW:]
        semsi = sems[:RSRC]
        semdi = sems[RSRC:RSRC + RDST]
        semr = sems[RSRC + RDST:RSRC + RDST + RROW]
        semw = sems[RSRC + RDST + RROW:]
        cid = lax.axis_index("c")
        sid = lax.axis_index("s")
        rbase = sid * ROWS_PER_TILE
        tb = sid * NCHUNK

        def pref_s(c, s):
            pltpu.async_copy(edges_hbm.at[2 * cid + 1, tb + c, 0],
                             sidx[s], semsi[s])

        def pref_d(c, d):
            pltpu.async_copy(edges_hbm.at[2 * cid, tb + c, 0],
                             didx[d], semdi[d])

        def wait_si(s):
            pltpu.make_async_copy(edges_hbm.at[0, 0, 0],
                                  sidx[s], semsi[s]).wait()

        def wait_di(d):
            pltpu.make_async_copy(edges_hbm.at[0, 0, 0],
                                  didx[d], semdi[d]).wait()

        def wait_rows(sem, r):
            pltpu.make_async_copy(x_hbm.at[pl.ds(0, CHUNK)],
                                  rows[r], sem[r]).wait()

        def gather(s, r):
            pltpu.async_copy(x_hbm.at[sidx[s]], rows[r], semr[r])

        def scatter(r, d):
            pltpu.async_copy(rows[r], acc.at[didx[d]], semw[r], add=True)

        for k in range(RSRC):
            pref_s(k, k)
        for k in range(G * 3):
            pref_d(k, k)

        @pl.when(sid < NSUB - 1)
        def _():
            pltpu.sync_copy(zeros_hbm, acc.at[pl.ds(rbase, ROWS_PER_TILE)])

        @pl.when(sid == NSUB - 1)
        def _():
            pltpu.sync_copy(zeros_hbm.at[pl.ds(0, ROWS_LAST)],
                            acc.at[pl.ds(rbase, ROWS_LAST)])

        plsc.subcore_barrier()
        for k in range(G):
            wait_si(k)
            gather(k, k)

        def step(c, u, first=False, pref=(True, True), do_gather=True):
            if not first:
                wait_rows(semw, (u + G) % RROW)
            if do_gather:
                wait_si((u + G) % RSRC)
                gather((u + G) % RSRC, (u + G) % RROW)
            wait_rows(semr, u % RROW)
            wait_di(u % RDST)
            scatter(u % RROW, u % RDST)
            if pref[0]:
                pref_s(c + RSRC, u % RSRC)
            if pref[1]:
                pref_d(c + RDST - G, (u - G) % RDST)

        for c in range(G):
            step(c, c, first=True)

        nsteady = (NCHUNK - 2 * G) // UNROLL

        def chunk_body(g, carry):
            for u0 in range(UNROLL):
                step(UNROLL * g + u0 + G, (u0 + G) % UNROLL)
            return carry

        lax.fori_loop(0, nsteady, chunk_body, 0)

        for c in range(G + UNROLL * nsteady, NCHUNK):
            step(c, c % UNROLL,
                 pref=(c + RSRC < NCHUNK, c + RDST - G < NCHUNK),
                 do_gather=c + G < NCHUNK)
        for c in range(NCHUNK - G, NCHUNK):
            wait_rows(semw, c % RROW)
        plsc.subcore_barrier()

        @pl.when(sid < NSUB - 1)
        def _():
            pltpu.sync_copy(acc.at[pl.ds(rbase, ROWS_PER_TILE)],
                            out_hbm.at[cid, pl.ds(rbase, ROWS_PER_TILE)])

        @pl.when(sid == NSUB - 1)
        def _():
            pltpu.sync_copy(acc.at[pl.ds(rbase, ROWS_LAST)],
                            out_hbm.at[cid, pl.ds(rbase, ROWS_LAST)])

    return _spmm_pair


BLK = 1000
NBLK = N // BLK


def _pre_body(feats_ref, wa_ref, wb_ref, w1_ref, b1_ref, w2_ref, o_ref):
    pid = pl.program_id(0)
    w = jnp.where(pid * BLK < N // 2, wa_ref[...], wb_ref[...])
    h = jnp.dot(feats_ref[...], w, preferred_element_type=jnp.float32)
    z = jnp.maximum(
        jnp.dot(h, w1_ref[...], preferred_element_type=jnp.float32)
        + b1_ref[...], 0.0)
    mean = jnp.mean(z, axis=1, keepdims=True)
    zc = z - mean
    std = jnp.sqrt(jnp.sum(zc * zc, axis=1, keepdims=True) / (D - 1))
    xn = jnp.where(std > 0.0, zc / std, 0.0)
    o_ref[...] = jnp.dot(xn, w2_ref[...], preferred_element_type=jnp.float32)


def _pre(feats, wa, wb, w1, b1, w2):
    return pl.pallas_call(
        _pre_body,
        grid=(NBLK,),
        in_specs=[
            pl.BlockSpec((BLK, D), lambda i: (i, 0)),
            pl.BlockSpec((D, D), lambda i: (0, 0)),
            pl.BlockSpec((D, D), lambda i: (0, 0)),
            pl.BlockSpec((D, D), lambda i: (0, 0)),
            pl.BlockSpec((1, D), lambda i: (0, 0)),
            pl.BlockSpec((D, F), lambda i: (0, 0)),
        ],
        out_specs=pl.BlockSpec((BLK, F), lambda i: (i, 0)),
        out_shape=jax.ShapeDtypeStruct((N, F), jnp.float32),
    )(feats, wa, wb, w1, b1, w2)


def _comb_body(wc_ref, x_ref, p1_ref, p2_ref, o_ref):
    o_ref[...] = (wc_ref[0] * x_ref[...]
                  + wc_ref[1] * p1_ref[0, 0] + wc_ref[2] * p1_ref[0, 1]
                  + wc_ref[3] * p2_ref[0, 0] + wc_ref[4] * p2_ref[0, 1])


def _fin_body(wc_ref, y_ref, p1_ref, p2_ref, b2_ref, o_ref):
    o_ref[...] = (wc_ref[0] * y_ref[...]
                  + wc_ref[1] * p1_ref[0, 1] + wc_ref[2] * p1_ref[0, 0]
                  + wc_ref[3] * p2_ref[0, 1] + wc_ref[4] * p2_ref[0, 0]
                  + b2_ref[...])


def _poly_combine(body, wcoef, x, pair1, pair2, *extra):
    extra_specs = [pl.BlockSpec((1, F), lambda i: (0, 0))] * len(extra)
    return pl.pallas_call(
        body,
        grid=(NBLK,),
        in_specs=[
            pl.BlockSpec(memory_space=pltpu.SMEM),
            pl.BlockSpec((BLK, F), lambda i: (i, 0)),
            pl.BlockSpec((1, 2, BLK, F), lambda i: (0, 0, i, 0)),
            pl.BlockSpec((1, 2, BLK, F), lambda i: (0, 0, i, 0)),
        ] + extra_specs,
        out_specs=pl.BlockSpec((BLK, F), lambda i: (i, 0)),
        out_shape=jax.ShapeDtypeStruct((N, F), jnp.float32),
    )(wcoef, x, pair1.reshape(1, 2, N, F), pair2.reshape(1, 2, N, F), *extra)


def kernel(feat_A, feat_B, edge_AB, edge_BA, Wproj_A, Wproj_B,
           lin1_W, lin1_b, lin2_W, lin2_b, Wcoef):
    feats = jnp.concatenate([feat_A, feat_B], axis=0)
    edges = (jnp.stack([edge_AB, edge_BA]).astype(jnp.int32)
             .reshape(4, NSUB * NCHUNK, 1, CHUNK))
    zeros = jnp.zeros((ROWS_PER_TILE, F), jnp.float32)

    z = _pre(feats, Wproj_A, Wproj_B, lin1_W, lin1_b.reshape(1, D), lin2_W)

    spmm_pair = _get_spmm_pair()
    pa = spmm_pair(z, edges, zeros)
    pb = spmm_pair(pa[1], edges, zeros)
    y = _poly_combine(_comb_body, Wcoef, z, pa, pb)

    pc = spmm_pair(y, edges, zeros)
    pd = spmm_pair(pc[0], edges, zeros)
    return _poly_combine(_fin_body, Wcoef, y, pc, pd, lin2_b.reshape(1, F))


# speedup vs baseline: 42.7478x; 1.1203x over previous
import functools

import jax
import jax.numpy as jnp
from jax import lax
from jax.experimental import pallas as pl
from jax.experimental.pallas import tpu as pltpu
from jax.experimental.pallas import tpu_sc as plsc

N = 10000
D = 128
F = 16
E = 256000
NCORES = 2
NSUB = 16
ROWS_PER_TILE = 640
ROWS_LAST = N - (NSUB - 1) * ROWS_PER_TILE
EDGES_PER_TILE = E // NSUB
CHUNK = 640
NCHUNK = EDGES_PER_TILE // CHUNK
RROW = 6
G = 3
RSRC = 6
RDST = 12
UNROLL = 12


@functools.lru_cache(maxsize=None)
def _get_spmm_pair(swap, xrows):
    mesh = plsc.VectorSubcoreMesh(
        core_axis_name="c", subcore_axis_name="s",
        num_cores=NCORES, num_subcores=NSUB)

    @functools.partial(
        pl.kernel,
        out_type=jax.ShapeDtypeStruct((NCORES, N, F), jnp.float32),
        mesh=mesh,
        scratch_types=(
            [pltpu.VMEM_SHARED((N, F), jnp.float32)]
            + [pltpu.VMEM((CHUNK,), jnp.int32) for _ in range(RSRC)]
            + [pltpu.VMEM((CHUNK,), jnp.int32) for _ in range(RDST)]
            + [pltpu.VMEM((CHUNK, F), jnp.float32) for _ in range(RROW)]
            + [pltpu.SemaphoreType.DMA for _ in range(RSRC + RDST + 2 * RROW)]
        ),
        compiler_params=pltpu.CompilerParams(use_tc_tiling_on_sc=False),
    )
    def _spmm_pair(x_hbm, eab_hbm, eba_hbm, zeros_hbm, out_hbm, acc, *bufs):
        sidx = bufs[:RSRC]
        didx = bufs[RSRC:RSRC + RDST]
        rows = bufs[RSRC + RDST:RSRC + RDST + RROW]
        sems = bufs[RSRC + RDST + RROW:]
        semsi = sems[:RSRC]
        semdi = sems[RSRC:RSRC + RDST]
        semr = sems[RSRC + RDST:RSRC + RDST + RROW]
        semw = sems[RSRC + RDST + RROW:]
        cid = lax.axis_index("c")
        sid = lax.axis_index("s")
        rbase = sid * ROWS_PER_TILE
        tb = sid * NCHUNK

        @pl.when(sid < NSUB - 1)
        def _():
            pltpu.sync_copy(zeros_hbm, acc.at[pl.ds(rbase, ROWS_PER_TILE)])

        @pl.when(sid == NSUB - 1)
        def _():
            pltpu.sync_copy(zeros_hbm.at[pl.ds(0, ROWS_LAST)],
                            acc.at[pl.ds(rbase, ROWS_LAST)])

        def pipeline(er):
            def pref_s(c, s):
                pltpu.async_copy(er.at[1, tb + c, 0], sidx[s], semsi[s])

            def pref_d(c, d):
                pltpu.async_copy(er.at[0, tb + c, 0], didx[d], semdi[d])

            def wait_si(s):
                pltpu.make_async_copy(er.at[0, 0, 0],
                                      sidx[s], semsi[s]).wait()

            def wait_di(d):
                pltpu.make_async_copy(er.at[0, 0, 0],
                                      didx[d], semdi[d]).wait()

            def wait_rows(sem, r):
                pltpu.make_async_copy(x_hbm.at[pl.ds(0, CHUNK)],
                                      rows[r], sem[r]).wait()

            def gather(s, r):
                pltpu.async_copy(x_hbm.at[sidx[s]], rows[r], semr[r])

            def scatter(r, d):
                pltpu.async_copy(rows[r], acc.at[didx[d]], semw[r], add=True)

            for k in range(RSRC):
                pref_s(k, k)
            for k in range(G * 3):
                pref_d(k, k)
            plsc.subcore_barrier()
            for k in range(G):
                wait_si(k)
                gather(k, k)

            def step(c, u, first=False, pref=(True, True), do_gather=True):
                if not first:
                    wait_rows(semw, (u + G) % RROW)
                if do_gather:
                    wait_si((u + G) % RSRC)
                    gather((u + G) % RSRC, (u + G) % RROW)
                wait_rows(semr, u % RROW)
                wait_di(u % RDST)
                scatter(u % RROW, u % RDST)
                if pref[0]:
                    pref_s(c + RSRC, u % RSRC)
                if pref[1]:
                    pref_d(c + RDST - G, (u - G) % RDST)

            for c in range(G):
                step(c, c, first=True)

            nsteady = (NCHUNK - 2 * G) // UNROLL

            def chunk_body(g, carry):
                for u0 in range(UNROLL):
                    step(UNROLL * g + u0 + G, (u0 + G) % UNROLL)
                return carry

            lax.fori_loop(0, nsteady, chunk_body, 0)

            for c in range(G + UNROLL * nsteady, NCHUNK):
                step(c, c % UNROLL,
                     pref=(c + RSRC < NCHUNK, c + RDST - G < NCHUNK),
                     do_gather=c + G < NCHUNK)
            for c in range(NCHUNK - G, NCHUNK):
                wait_rows(semw, c % RROW)

        @pl.when(cid == 0)
        def _():
            pipeline(eab_hbm)

        @pl.when(cid == 1)
        def _():
            pipeline(eba_hbm)

        plsc.subcore_barrier()
        oid = (1 - cid) if swap else cid

        @pl.when(sid < NSUB - 1)
        def _():
            pltpu.sync_copy(acc.at[pl.ds(rbase, ROWS_PER_TILE)],
                            out_hbm.at[oid, pl.ds(rbase, ROWS_PER_TILE)])

        @pl.when(sid == NSUB - 1)
        def _():
            pltpu.sync_copy(acc.at[pl.ds(rbase, ROWS_LAST)],
                            out_hbm.at[oid, pl.ds(rbase, ROWS_LAST)])

    def wrapped(x, eab, eba, zeros):
        assert x.shape == (xrows, F)
        return _spmm_pair(x, eab, eba, zeros)

    return wrapped


BLK = 1000
NBLK = N // BLK
HBLK = NBLK // 2


def _pre_body(fa_ref, fb_ref, wa_ref, wb_ref, w1_ref, b1_ref, w2_ref, o_ref):
    pid = pl.program_id(0)
    first = pid < HBLK
    w = jnp.where(first, wa_ref[...], wb_ref[...])
    feats = jnp.where(first, fa_ref[...], fb_ref[...])
    h = jnp.dot(feats, w, preferred_element_type=jnp.float32)
    z = jnp.maximum(
        jnp.dot(h, w1_ref[...], preferred_element_type=jnp.float32)
        + b1_ref[...], 0.0)
    mean = jnp.mean(z, axis=1, keepdims=True)
    zc = z - mean
    std = jnp.sqrt(jnp.sum(zc * zc, axis=1, keepdims=True) / (D - 1))
    xn = jnp.where(std > 0.0, zc / std, 0.0)
    o_ref[...] = jnp.dot(xn, w2_ref[...], preferred_element_type=jnp.float32)


def _pre(fa, fb, wa, wb, w1, b1, w2):
    return pl.pallas_call(
        _pre_body,
        grid=(NBLK,),
        in_specs=[
            pl.BlockSpec((BLK, D), lambda i: (i % HBLK, 0)),
            pl.BlockSpec((BLK, D), lambda i: (i % HBLK, 0)),
            pl.BlockSpec((D, D), lambda i: (0, 0)),
            pl.BlockSpec((D, D), lambda i: (0, 0)),
            pl.BlockSpec((D, D), lambda i: (0, 0)),
            pl.BlockSpec((1, D), lambda i: (0, 0)),
            pl.BlockSpec((D, F), lambda i: (0, 0)),
        ],
        out_specs=pl.BlockSpec((BLK, F), lambda i: (i, 0)),
        out_shape=jax.ShapeDtypeStruct((N, F), jnp.float32),
    )(fa, fb, wa, wb, w1, b1, w2)


def _comb_body(wc_ref, x_ref, p1_ref, p2_ref, o_ref):
    o_ref[...] = (wc_ref[0] * x_ref[...]
                  + wc_ref[1] * p1_ref[1] + wc_ref[2] * p1_ref[0]
                  + wc_ref[3] * p2_ref[0] + wc_ref[4] * p2_ref[1])


def _fin_body(wc_ref, y_ref, p1_ref, p2_ref, b2_ref, o_ref):
    o_ref[...] = (wc_ref[0] * y_ref[...]
                  + wc_ref[1] * p1_ref[1] + wc_ref[2] * p1_ref[0]
                  + wc_ref[3] * p2_ref[1] + wc_ref[4] * p2_ref[0]
                  + b2_ref[...])


def _poly_combine(body, wcoef, x, pair1, pair2, *extra):
    extra_specs = [pl.BlockSpec((1, F), lambda i: (0, 0))] * len(extra)
    return pl.pallas_call(
        body,
        grid=(NBLK,),
        in_specs=[
            pl.BlockSpec(memory_space=pltpu.SMEM),
            pl.BlockSpec((BLK, F), lambda i: (i, 0)),
            pl.BlockSpec((2, BLK, F), lambda i: (0, i, 0)),
            pl.BlockSpec((2, BLK, F), lambda i: (0, i, 0)),
        ] + extra_specs,
        out_specs=pl.BlockSpec((BLK, F), lambda i: (i, 0)),
        out_shape=jax.ShapeDtypeStruct((N, F), jnp.float32),
    )(wcoef, x, pair1, pair2, *extra)


def kernel(feat_A, feat_B, edge_AB, edge_BA, Wproj_A, Wproj_B,
           lin1_W, lin1_b, lin2_W, lin2_b, Wcoef):
    eab = edge_AB.reshape(2, NSUB * NCHUNK, 1, CHUNK)
    eba = edge_BA.reshape(2, NSUB * NCHUNK, 1, CHUNK)
    zeros = jnp.zeros((ROWS_PER_TILE, F), jnp.float32)

    z = _pre(feat_A, feat_B, Wproj_A, Wproj_B,
             lin1_W, lin1_b.reshape(1, D), lin2_W)

    pa = _get_spmm_pair(True, N)(z, eab, eba, zeros)
    pb = _get_spmm_pair(False, 2 * N)(
        pa.reshape(2 * N, F), eab, eba, zeros)
    y = _poly_combine(_comb_body, Wcoef, z, pa, pb)

    pc = _get_spmm_pair(False, N)(y, eab, eba, zeros)
    pd = _get_spmm_pair(False, 2 * N)(
        pc.reshape(2 * N, F), eab, eba, zeros)
    return _poly_combine(_fin_body, Wcoef, y, pc, pd, lin2_b.reshape(1, F))
